# speedup vs baseline: 2.1065x; 2.1065x over previous
import functools

import jax
import jax.numpy as jnp
from jax import lax
from jax.experimental import pallas as pl
from jax.experimental.pallas import tpu as pltpu
from jax.experimental.pallas import tpu_sc as plsc

N_EDGES = 320000
D = 128
NC = 2
NS = 16
NW = NC * NS
B_PER_W = N_EDGES // NW
C = 80
CHUNKS = B_PER_W // C
LANES = 16


def _edge_sym_body(tbl, src, dst, out, sidx, didx, ti, tj, ob, sem1, sem2):
    wid = lax.axis_index("s") * NC + lax.axis_index("c")
    base = wid * B_PER_W

    def chunk_body(k, carry):
        off = base + k * C
        pltpu.sync_copy(src.at[pl.ds(off, C)], sidx)
        pltpu.sync_copy(dst.at[pl.ds(off, C)], didx)
        cp1 = pltpu.async_copy(tbl.at[sidx], ti, sem1)
        cp2 = pltpu.async_copy(tbl.at[didx], tj, sem2)
        cp1.wait()
        cp2.wait()

        def row_body(i, c2):
            for j in range(D // LANES):
                a = ti[i, pl.ds(j * LANES, LANES)]
                b = tj[i, pl.ds(j * LANES, LANES)]
                ob[i, pl.ds(j * LANES, LANES)] = a + b
                ob[i, pl.ds(D + j * LANES, LANES)] = a - b
            return c2

        lax.fori_loop(0, C, row_body, 0)
        pltpu.sync_copy(ob, out.at[pl.ds(off, C)])
        return carry

    lax.fori_loop(0, CHUNKS, chunk_body, 0)


_edge_sym = functools.partial(
    pl.kernel,
    mesh=plsc.VectorSubcoreMesh(core_axis_name="c", subcore_axis_name="s"),
    out_type=jax.ShapeDtypeStruct((N_EDGES, 2 * D), jnp.float32),
    scratch_types=[
        pltpu.VMEM((C,), jnp.int32),
        pltpu.VMEM((C,), jnp.int32),
        pltpu.VMEM((C, D), jnp.float32),
        pltpu.VMEM((C, D), jnp.float32),
        pltpu.VMEM((C, 2 * D), jnp.float32),
        pltpu.SemaphoreType.DMA,
        pltpu.SemaphoreType.DMA,
    ],
)(_edge_sym_body)


def kernel(node_attrs, edge_index):
    ei = edge_index.astype(jnp.int32)
    return _edge_sym(node_attrs, ei[0], ei[1])


# speedup vs baseline: 3.7777x; 1.7933x over previous
import functools

import jax
import jax.numpy as jnp
from jax import lax
from jax.experimental import pallas as pl
from jax.experimental.pallas import tpu as pltpu
from jax.experimental.pallas import tpu_sc as plsc

N_EDGES = 320000
D = 128
NC = 2
NS = 16
NW = NC * NS
B_PER_W = N_EDGES // NW
C = 40
CHUNKS = B_PER_W // C
NBUF = 2
LANES = 16


def _edge_sym_body(tbl, src, dst, out, sidx, didx,
                   ti0, ti1, tj0, tj1, ob0, ob1, gs0, gs1, ss0, ss1):
    ti = [ti0, ti1]
    tj = [tj0, tj1]
    ob = [ob0, ob1]
    gsem = [gs0, gs1]
    ssem = [ss0, ss1]

    wid = lax.axis_index("s") * NC + lax.axis_index("c")
    base = wid * B_PER_W
    pltpu.sync_copy(src.at[pl.ds(base, B_PER_W)], sidx)
    pltpu.sync_copy(dst.at[pl.ds(base, B_PER_W)], didx)

    def gather_copies(b, g):
        lo = g * C
        c1 = pltpu.make_async_copy(tbl.at[sidx.at[pl.ds(lo, C)]], ti[b], gsem[b])
        c2 = pltpu.make_async_copy(tbl.at[didx.at[pl.ds(lo, C)]], tj[b], gsem[b])
        return c1, c2

    def issue_gathers(b, g):
        c1, c2 = gather_copies(b, g)
        c1.start()
        c2.start()

    def wait_gathers(b, g):
        c1, c2 = gather_copies(b, g)
        c1.wait()
        c2.wait()

    def issue_store(b, g):
        off = base + g * C
        pltpu.make_async_copy(ob[b], out.at[pl.ds(off, C)], ssem[b]).start()

    def wait_store(b):
        pltpu.make_async_copy(ob[b], out.at[pl.ds(base, C)], ssem[b]).wait()

    issue_gathers(0, 0)
    issue_gathers(1, 1)

    def outer(g2, carry):
        for b in range(NBUF):
            g = g2 * NBUF + b
            wait_gathers(b, g)

            @pl.when(g2 > 0)
            def _():
                wait_store(b)

            def row_body(i, c2):
                for j in range(D // LANES):
                    a = ti[b][i, pl.ds(j * LANES, LANES)]
                    bb = tj[b][i, pl.ds(j * LANES, LANES)]
                    ob[b][i, pl.ds(j * LANES, LANES)] = a + bb
                    ob[b][i, pl.ds(D + j * LANES, LANES)] = a - bb
                return c2

            lax.fori_loop(0, C, row_body, 0)
            issue_store(b, g)

            @pl.when(g2 < CHUNKS // NBUF - 1)
            def _():
                issue_gathers(b, g + NBUF)
        return carry

    lax.fori_loop(0, CHUNKS // NBUF, outer, 0)
    for b in range(NBUF):
        wait_store(b)


_edge_sym = functools.partial(
    pl.kernel,
    mesh=plsc.VectorSubcoreMesh(core_axis_name="c", subcore_axis_name="s"),
    out_type=jax.ShapeDtypeStruct((N_EDGES, 2 * D), jnp.float32),
    scratch_types=[
        pltpu.VMEM((B_PER_W,), jnp.int32),
        pltpu.VMEM((B_PER_W,), jnp.int32),
        pltpu.VMEM((C, D), jnp.float32),
        pltpu.VMEM((C, D), jnp.float32),
        pltpu.VMEM((C, D), jnp.float32),
        pltpu.VMEM((C, D), jnp.float32),
        pltpu.VMEM((C, 2 * D), jnp.float32),
        pltpu.VMEM((C, 2 * D), jnp.float32),
        pltpu.SemaphoreType.DMA,
        pltpu.SemaphoreType.DMA,
        pltpu.SemaphoreType.DMA,
        pltpu.SemaphoreType.DMA,
    ],
)(_edge_sym_body)


def kernel(node_attrs, edge_index):
    ei = edge_index.astype(jnp.int32)
    return _edge_sym(node_attrs, ei[0], ei[1])
